# baseline (speedup 1.0000x reference)
import functools

import jax
import jax.numpy as jnp
from jax import lax
from jax.experimental import pallas as pl
from jax.experimental.pallas import tpu as pltpu
from jax.experimental.pallas import tpu_sc as plsc

_NUM_WORKERS = 32
_CHUNK = 128
_NBUF = 4


def _sc_gather(table, idx_flat):
    n = idx_flat.shape[0]
    d = table.shape[1]
    per_worker = _CHUNK * _NBUF
    n_pad = ((n + _NUM_WORKERS * per_worker - 1)
             // (_NUM_WORKERS * per_worker)) * (_NUM_WORKERS * per_worker)
    idx_pad = jnp.zeros((n_pad,), jnp.int32).at[:n].set(idx_flat)
    idx2d = idx_pad.reshape(n_pad // _CHUNK, _CHUNK)
    nch = (n_pad // _CHUNK) // _NUM_WORKERS

    mesh = plsc.VectorSubcoreMesh(core_axis_name="c", subcore_axis_name="s")

    @functools.partial(
        pl.kernel,
        out_type=jax.ShapeDtypeStruct((n_pad, d), jnp.float32),
        mesh=mesh,
        scratch_types=(
            [pltpu.VMEM((nch, _CHUNK), jnp.int32),
             pltpu.VMEM((_NBUF, _CHUNK, d), jnp.float32)]
            + [pltpu.SemaphoreType.DMA] * _NBUF
        ),
        compiler_params=pltpu.CompilerParams(use_tc_tiling_on_sc=False),
    )
    def gather_kernel(table_hbm, idx_hbm, out_hbm, idx_v, rows_v, *gsems):
        wid = lax.axis_index("c") * 16 + lax.axis_index("s")
        chunk0 = wid * nch
        row0 = chunk0 * _CHUNK
        pltpu.sync_copy(idx_hbm.at[pl.ds(chunk0, nch)], idx_v)

        def fire(j, b):
            pltpu.make_async_copy(
                table_hbm.at[idx_v.at[j]], rows_v.at[b], gsems[b]).start()

        for b in range(_NBUF):
            fire(b, b)

        def group(g, _):
            for b in range(_NBUF):
                j = g * _NBUF + b
                pltpu.make_async_copy(
                    table_hbm.at[idx_v.at[j]], rows_v.at[b], gsems[b]).wait()
                pltpu.sync_copy(rows_v.at[b],
                                out_hbm.at[pl.ds(row0 + j * _CHUNK, _CHUNK)])

                @pl.when(j + _NBUF < nch)
                def _():
                    fire(j + _NBUF, b)
            return 0

        lax.fori_loop(0, nch // _NBUF, group, 0)

    return gather_kernel(table, idx2d)[:n]


def _mlp_block(h_ref, w1_ref, b1_ref, w2_ref, b2_ref, o_ref):
    h = h_ref[...]
    z = jnp.dot(h, w1_ref[...], preferred_element_type=jnp.float32,
                precision=lax.Precision.HIGHEST) + b1_ref[...]
    z = jnp.maximum(z, 0.0)
    o_ref[...] = jnp.dot(z, w2_ref[...], preferred_element_type=jnp.float32,
                         precision=lax.Precision.HIGHEST) + b2_ref[...]


def _tc_mlp(h, w1, b1, w2, b2, block_rows=2048):
    n, d = h.shape
    h1 = w1.shape[1]
    d2 = w2.shape[1]
    n_pad = ((n + block_rows - 1) // block_rows) * block_rows
    if n_pad != n:
        h = jnp.zeros((n_pad, d), h.dtype).at[:n].set(h)
    out = pl.pallas_call(
        _mlp_block,
        grid=(n_pad // block_rows,),
        in_specs=[
            pl.BlockSpec((block_rows, d), lambda i: (i, 0)),
            pl.BlockSpec((d, h1), lambda i: (0, 0)),
            pl.BlockSpec((1, h1), lambda i: (0, 0)),
            pl.BlockSpec((h1, d2), lambda i: (0, 0)),
            pl.BlockSpec((1, d2), lambda i: (0, 0)),
        ],
        out_specs=pl.BlockSpec((block_rows, d2), lambda i: (i, 0)),
        out_shape=jax.ShapeDtypeStruct((n_pad, d2), jnp.float32),
    )(h, w1, b1.reshape(1, h1), w2, b2.reshape(1, d2))
    return out[:n]


def kernel(x, table, W1, b1, W2, b2):
    batch, fields = x.shape
    idx_flat = x.reshape(-1).astype(jnp.int32)
    gathered = _sc_gather(table, idx_flat)
    out = _tc_mlp(gathered, W1, b1, W2, b2)
    return out.reshape(batch, fields, W2.shape[1])


# speedup vs baseline: 1.1103x; 1.1103x over previous
import functools

import jax
import jax.numpy as jnp
from jax import lax
from jax.experimental import pallas as pl
from jax.experimental.pallas import tpu as pltpu
from jax.experimental.pallas import tpu_sc as plsc

_NUM_WORKERS = 32
_CHUNK = 128
_NBUF = 4


def _repack_block(t_ref, o_ref):
    o_ref[...] = t_ref[...].T


def _tc_repack(table_t, block_cols=4096):
    d, vocab = table_t.shape
    grid = (vocab + block_cols - 1) // block_cols
    return pl.pallas_call(
        _repack_block,
        grid=(grid,),
        in_specs=[pl.BlockSpec((d, block_cols), lambda j: (0, j))],
        out_specs=pl.BlockSpec((block_cols, d), lambda j: (j, 0)),
        out_shape=jax.ShapeDtypeStruct((vocab, d), jnp.float32),
    )(table_t)


def _sc_gather(table, idx_flat):
    n = idx_flat.shape[0]
    d = table.shape[1]
    per_worker = _CHUNK * _NBUF
    n_pad = ((n + _NUM_WORKERS * per_worker - 1)
             // (_NUM_WORKERS * per_worker)) * (_NUM_WORKERS * per_worker)
    idx_pad = jnp.zeros((n_pad,), jnp.int32).at[:n].set(idx_flat)
    idx2d = idx_pad.reshape(n_pad // _CHUNK, _CHUNK)
    nch = (n_pad // _CHUNK) // _NUM_WORKERS

    mesh = plsc.VectorSubcoreMesh(core_axis_name="c", subcore_axis_name="s")

    @functools.partial(
        pl.kernel,
        out_type=jax.ShapeDtypeStruct((n_pad, d), jnp.float32),
        mesh=mesh,
        scratch_types=(
            [pltpu.VMEM((nch, _CHUNK), jnp.int32),
             pltpu.VMEM((_NBUF, _CHUNK, d), jnp.float32)]
            + [pltpu.SemaphoreType.DMA] * _NBUF
        ),
        compiler_params=pltpu.CompilerParams(use_tc_tiling_on_sc=False),
    )
    def gather_kernel(table_hbm, idx_hbm, out_hbm, idx_v, rows_v, *gsems):
        wid = lax.axis_index("c") * 16 + lax.axis_index("s")
        chunk0 = wid * nch
        row0 = chunk0 * _CHUNK
        pltpu.sync_copy(idx_hbm.at[pl.ds(chunk0, nch)], idx_v)

        def fire(j, b):
            pltpu.make_async_copy(
                table_hbm.at[idx_v.at[j]], rows_v.at[b], gsems[b]).start()

        for b in range(_NBUF):
            fire(b, b)

        def group(g, _):
            for b in range(_NBUF):
                j = g * _NBUF + b
                pltpu.make_async_copy(
                    table_hbm.at[idx_v.at[j]], rows_v.at[b], gsems[b]).wait()
                pltpu.sync_copy(rows_v.at[b],
                                out_hbm.at[pl.ds(row0 + j * _CHUNK, _CHUNK)])

                @pl.when(j + _NBUF < nch)
                def _():
                    fire(j + _NBUF, b)
            return 0

        lax.fori_loop(0, nch // _NBUF, group, 0)

    return gather_kernel(table, idx2d)[:n]


def _mlp_block_t(h_ref, w1_ref, b1_ref, w2_ref, b2_ref, o_ref):
    h = h_ref[...]
    z = jnp.dot(h, w1_ref[...], preferred_element_type=jnp.float32,
                precision=lax.Precision.HIGHEST) + b1_ref[...]
    z = jnp.maximum(z, 0.0)
    o = jnp.dot(z, w2_ref[...], preferred_element_type=jnp.float32,
                precision=lax.Precision.HIGHEST) + b2_ref[...]
    o_ref[0] = o.T


def _tc_mlp_t(g, w1, b1, w2, b2, fields, batch, block_rows=2048):
    d = g.shape[1]
    h1 = w1.shape[1]
    d2 = w2.shape[1]
    nb = batch // block_rows
    out = pl.pallas_call(
        _mlp_block_t,
        grid=(fields, nb),
        in_specs=[
            pl.BlockSpec((block_rows, d), lambda f, j: (f * nb + j, 0)),
            pl.BlockSpec((d, h1), lambda f, j: (0, 0)),
            pl.BlockSpec((1, h1), lambda f, j: (0, 0)),
            pl.BlockSpec((h1, d2), lambda f, j: (0, 0)),
            pl.BlockSpec((1, d2), lambda f, j: (0, 0)),
        ],
        out_specs=pl.BlockSpec((1, d2, block_rows), lambda f, j: (f, 0, j)),
        out_shape=jax.ShapeDtypeStruct((fields, d2, batch), jnp.float32),
    )(g, w1, b1.reshape(1, h1), w2, b2.reshape(1, d2))
    return out


def kernel(x, table, W1, b1, W2, b2):
    batch, fields = x.shape
    idx_flat = x.T.reshape(-1).astype(jnp.int32)
    table_rm = _tc_repack(table.T)
    gathered = _sc_gather(table_rm, idx_flat)
    out_t = _tc_mlp_t(gathered, W1, b1, W2, b2, fields, batch)
    return out_t.transpose(2, 0, 1)


# speedup vs baseline: 1.5394x; 1.3864x over previous
import functools

import jax
import jax.numpy as jnp
from jax import lax
from jax.experimental import pallas as pl
from jax.experimental.pallas import tpu as pltpu
from jax.experimental.pallas import tpu_sc as plsc

_NUM_WORKERS = 32
_CHUNK = 128
_NBUF = 4


def _repack_block(t_ref, o_ref):
    o_ref[...] = t_ref[...].T


def _tc_repack(table_t, block_cols=4096):
    d, vocab = table_t.shape
    grid = (vocab + block_cols - 1) // block_cols
    return pl.pallas_call(
        _repack_block,
        grid=(grid,),
        in_specs=[pl.BlockSpec((d, block_cols), lambda j: (0, j))],
        out_specs=pl.BlockSpec((block_cols, d), lambda j: (j, 0)),
        out_shape=jax.ShapeDtypeStruct((vocab, d), jnp.float32),
    )(table_t)


def _sc_gather(table, idx_flat):
    n = idx_flat.shape[0]
    d = table.shape[1]
    per_worker = _CHUNK * _NBUF
    n_pad = ((n + _NUM_WORKERS * per_worker - 1)
             // (_NUM_WORKERS * per_worker)) * (_NUM_WORKERS * per_worker)
    idx_pad = jnp.zeros((n_pad,), jnp.int32).at[:n].set(idx_flat)
    idx2d = idx_pad.reshape(n_pad // _CHUNK, _CHUNK)
    nch = (n_pad // _CHUNK) // _NUM_WORKERS

    mesh = plsc.VectorSubcoreMesh(core_axis_name="c", subcore_axis_name="s")

    @functools.partial(
        pl.kernel,
        out_type=jax.ShapeDtypeStruct((n_pad, d), jnp.float32),
        mesh=mesh,
        scratch_types=(
            [pltpu.VMEM((nch, _CHUNK), jnp.int32),
             pltpu.VMEM((_NBUF, _CHUNK, d), jnp.float32)]
            + [pltpu.SemaphoreType.DMA] * _NBUF
        ),
        compiler_params=pltpu.CompilerParams(use_tc_tiling_on_sc=False),
    )
    def gather_kernel(table_hbm, idx_hbm, out_hbm, idx_v, rows_v, *gsems):
        wid = lax.axis_index("c") * 16 + lax.axis_index("s")
        chunk0 = wid * nch
        row0 = chunk0 * _CHUNK
        pltpu.sync_copy(idx_hbm.at[pl.ds(chunk0, nch)], idx_v)

        def fire(j, b):
            pltpu.make_async_copy(
                table_hbm.at[idx_v.at[j]], rows_v.at[b], gsems[b]).start()

        for b in range(_NBUF):
            fire(b, b)

        def group(g, _):
            for b in range(_NBUF):
                j = g * _NBUF + b
                pltpu.make_async_copy(
                    table_hbm.at[idx_v.at[j]], rows_v.at[b], gsems[b]).wait()
                pltpu.sync_copy(rows_v.at[b],
                                out_hbm.at[pl.ds(row0 + j * _CHUNK, _CHUNK)])

                @pl.when(j + _NBUF < nch)
                def _():
                    fire(j + _NBUF, b)
            return 0

        lax.fori_loop(0, nch // _NBUF, group, 0)

    return gather_kernel(table, idx2d)[:n]


def _mlp_block_t(h_ref, w1_ref, b1_ref, w2_ref, b2_ref, o_ref):
    h = h_ref[...]
    z = jnp.dot(h, w1_ref[...],
                preferred_element_type=jnp.float32) + b1_ref[...]
    z = jnp.maximum(z, 0.0)
    o = jnp.dot(z, w2_ref[...],
                preferred_element_type=jnp.float32) + b2_ref[...]
    o_ref[0] = o.T


def _tc_mlp_t(g, w1, b1, w2, b2, fields, batch, block_rows=2048):
    d = g.shape[1]
    h1 = w1.shape[1]
    d2 = w2.shape[1]
    nb = batch // block_rows
    out = pl.pallas_call(
        _mlp_block_t,
        grid=(fields, nb),
        in_specs=[
            pl.BlockSpec((block_rows, d), lambda f, j: (f * nb + j, 0)),
            pl.BlockSpec((d, h1), lambda f, j: (0, 0)),
            pl.BlockSpec((1, h1), lambda f, j: (0, 0)),
            pl.BlockSpec((h1, d2), lambda f, j: (0, 0)),
            pl.BlockSpec((1, d2), lambda f, j: (0, 0)),
        ],
        out_specs=pl.BlockSpec((1, d2, block_rows), lambda f, j: (f, 0, j)),
        out_shape=jax.ShapeDtypeStruct((fields, d2, batch), jnp.float32),
    )(g, w1, b1.reshape(1, h1), w2, b2.reshape(1, d2))
    return out


def kernel(x, table, W1, b1, W2, b2):
    batch, fields = x.shape
    idx_flat = x.T.reshape(-1).astype(jnp.int32)
    table_rm = _tc_repack(table.T)
    gathered = _sc_gather(table_rm, idx_flat)
    out_t = _tc_mlp_t(gathered, W1, b1, W2, b2, fields, batch)
    return out_t.transpose(2, 0, 1)


# speedup vs baseline: 1.7195x; 1.1170x over previous
import functools

import jax
import jax.numpy as jnp
from jax import lax
from jax.experimental import pallas as pl
from jax.experimental.pallas import tpu as pltpu
from jax.experimental.pallas import tpu_sc as plsc

_NUM_WORKERS = 32
_CHUNK = 128
_NBUF = 4


def _repack_block(t_ref, o_ref):
    o_ref[...] = t_ref[...].T


def _tc_repack(table_t, block_cols=4096):
    d, vocab = table_t.shape
    grid = (vocab + block_cols - 1) // block_cols
    return pl.pallas_call(
        _repack_block,
        grid=(grid,),
        in_specs=[pl.BlockSpec((d, block_cols), lambda j: (0, j))],
        out_specs=pl.BlockSpec((block_cols, d), lambda j: (j, 0)),
        out_shape=jax.ShapeDtypeStruct((vocab, d), jnp.float32),
    )(table_t)


def _sc_gather(table, idx_flat):
    n = idx_flat.shape[0]
    d = table.shape[1]
    per_worker = _CHUNK * _NBUF
    n_pad = ((n + _NUM_WORKERS * per_worker - 1)
             // (_NUM_WORKERS * per_worker)) * (_NUM_WORKERS * per_worker)
    idx_pad = jnp.zeros((n_pad,), jnp.int32).at[:n].set(idx_flat)
    idx2d = idx_pad.reshape(n_pad // _CHUNK, _CHUNK)
    nch = (n_pad // _CHUNK) // _NUM_WORKERS

    mesh = plsc.VectorSubcoreMesh(core_axis_name="c", subcore_axis_name="s")

    @functools.partial(
        pl.kernel,
        out_type=jax.ShapeDtypeStruct((n_pad, d), jnp.float32),
        mesh=mesh,
        scratch_types=(
            [pltpu.VMEM((nch, _CHUNK), jnp.int32),
             pltpu.VMEM((_NBUF, _CHUNK, d), jnp.float32)]
            + [pltpu.SemaphoreType.DMA] * _NBUF
        ),
        compiler_params=pltpu.CompilerParams(use_tc_tiling_on_sc=False),
    )
    def gather_kernel(table_hbm, idx_hbm, out_hbm, idx_v, rows_v, *gsems):
        wid = lax.axis_index("c") * 16 + lax.axis_index("s")
        chunk0 = wid * nch
        row0 = chunk0 * _CHUNK
        pltpu.sync_copy(idx_hbm.at[pl.ds(chunk0, nch)], idx_v)

        def fire(j, b):
            pltpu.make_async_copy(
                table_hbm.at[idx_v.at[j]], rows_v.at[b], gsems[b]).start()

        for b in range(_NBUF):
            fire(b, b)

        def group(g, _):
            for b in range(_NBUF):
                j = g * _NBUF + b
                pltpu.make_async_copy(
                    table_hbm.at[idx_v.at[j]], rows_v.at[b], gsems[b]).wait()
                pltpu.sync_copy(rows_v.at[b],
                                out_hbm.at[pl.ds(row0 + j * _CHUNK, _CHUNK)])

                @pl.when(j + _NBUF < nch)
                def _():
                    fire(j + _NBUF, b)
            return 0

        lax.fori_loop(0, nch // _NBUF, group, 0)

    return gather_kernel(table, idx2d)[:n]


def _mlp_block_t(h_ref, w1_ref, b1_ref, w2_ref, b2_ref, o_ref):
    h = h_ref[...]
    z = jnp.dot(h, w1_ref[...],
                preferred_element_type=jnp.float32) + b1_ref[...]
    z = jnp.maximum(z, 0.0)
    o = jnp.dot(z, w2_ref[...],
                preferred_element_type=jnp.float32) + b2_ref[...]
    o_ref[0] = o.T


def _tc_mlp_t(g, w1, b1, w2, b2, fields, batch, block_rows=2048):
    d = g.shape[1]
    h1 = w1.shape[1]
    d2 = w2.shape[1]
    nb = batch // block_rows
    out = pl.pallas_call(
        _mlp_block_t,
        grid=(fields, nb),
        in_specs=[
            pl.BlockSpec((block_rows, d), lambda f, j: (f * nb + j, 0)),
            pl.BlockSpec((d, h1), lambda f, j: (0, 0)),
            pl.BlockSpec((1, h1), lambda f, j: (0, 0)),
            pl.BlockSpec((h1, d2), lambda f, j: (0, 0)),
            pl.BlockSpec((1, d2), lambda f, j: (0, 0)),
        ],
        out_specs=pl.BlockSpec((1, d2, block_rows), lambda f, j: (f, 0, j)),
        out_shape=jax.ShapeDtypeStruct((fields, d2, batch), jnp.float32),
    )(g, w1, b1.reshape(1, h1), w2, b2.reshape(1, d2))
    return out


def kernel(x, table, W1, b1, W2, b2):
    batch, fields = x.shape
    idx_flat = x.T.reshape(-1).astype(jnp.int32)
    gathered = _sc_gather(table, idx_flat)
    out_t = _tc_mlp_t(gathered, W1, b1, W2, b2, fields, batch)
    return out_t.transpose(2, 0, 1)


# speedup vs baseline: 1.8241x; 1.0608x over previous
import functools

import jax
import jax.numpy as jnp
from jax import lax
from jax.experimental import pallas as pl
from jax.experimental.pallas import tpu as pltpu
from jax.experimental.pallas import tpu_sc as plsc

_NUM_WORKERS = 32
_CHUNK = 128
_NBUF = 8


def _repack_block(t_ref, o_ref):
    o_ref[...] = t_ref[...].T


def _tc_repack(table_t, block_cols=4096):
    d, vocab = table_t.shape
    grid = (vocab + block_cols - 1) // block_cols
    return pl.pallas_call(
        _repack_block,
        grid=(grid,),
        in_specs=[pl.BlockSpec((d, block_cols), lambda j: (0, j))],
        out_specs=pl.BlockSpec((block_cols, d), lambda j: (j, 0)),
        out_shape=jax.ShapeDtypeStruct((vocab, d), jnp.float32),
    )(table_t)


def _sc_gather(table, idx_flat):
    n = idx_flat.shape[0]
    d = table.shape[1]
    per_worker = _CHUNK * _NBUF
    n_pad = ((n + _NUM_WORKERS * per_worker - 1)
             // (_NUM_WORKERS * per_worker)) * (_NUM_WORKERS * per_worker)
    idx_pad = jnp.zeros((n_pad,), jnp.int32).at[:n].set(idx_flat)
    idx2d = idx_pad.reshape(n_pad // _CHUNK, _CHUNK)
    nch = (n_pad // _CHUNK) // _NUM_WORKERS

    mesh = plsc.VectorSubcoreMesh(core_axis_name="c", subcore_axis_name="s")

    @functools.partial(
        pl.kernel,
        out_type=jax.ShapeDtypeStruct((n_pad, d), jnp.float32),
        mesh=mesh,
        scratch_types=(
            [pltpu.VMEM((nch, _CHUNK), jnp.int32),
             pltpu.VMEM((_NBUF, _CHUNK, d), jnp.float32)]
            + [pltpu.SemaphoreType.DMA] * _NBUF
        ),
        compiler_params=pltpu.CompilerParams(use_tc_tiling_on_sc=False),
    )
    def gather_kernel(table_hbm, idx_hbm, out_hbm, idx_v, rows_v, *gsems):
        wid = lax.axis_index("c") * 16 + lax.axis_index("s")
        chunk0 = wid * nch
        row0 = chunk0 * _CHUNK
        pltpu.sync_copy(idx_hbm.at[pl.ds(chunk0, nch)], idx_v)

        def fire(j, b):
            pltpu.make_async_copy(
                table_hbm.at[idx_v.at[j]], rows_v.at[b], gsems[b]).start()

        for b in range(_NBUF):
            fire(b, b)

        def group(g, _):
            for b in range(_NBUF):
                j = g * _NBUF + b
                pltpu.make_async_copy(
                    table_hbm.at[idx_v.at[j]], rows_v.at[b], gsems[b]).wait()
                pltpu.sync_copy(rows_v.at[b],
                                out_hbm.at[pl.ds(row0 + j * _CHUNK, _CHUNK)])

                @pl.when(j + _NBUF < nch)
                def _():
                    fire(j + _NBUF, b)
            return 0

        lax.fori_loop(0, nch // _NBUF, group, 0)

    return gather_kernel(table, idx2d)[:n]


def _mlp_block_t(h_ref, w1_ref, b1_ref, w2_ref, b2_ref, o_ref):
    h = h_ref[...]
    z = jnp.dot(h, w1_ref[...],
                preferred_element_type=jnp.float32) + b1_ref[...]
    z = jnp.maximum(z, 0.0)
    o = jnp.dot(z, w2_ref[...],
                preferred_element_type=jnp.float32) + b2_ref[...]
    o_ref[0] = o.T


def _tc_mlp_t(g, w1, b1, w2, b2, fields, batch, block_rows=4096):
    d = g.shape[1]
    h1 = w1.shape[1]
    d2 = w2.shape[1]
    nb = batch // block_rows
    out = pl.pallas_call(
        _mlp_block_t,
        grid=(fields, nb),
        in_specs=[
            pl.BlockSpec((block_rows, d), lambda f, j: (f * nb + j, 0)),
            pl.BlockSpec((d, h1), lambda f, j: (0, 0)),
            pl.BlockSpec((1, h1), lambda f, j: (0, 0)),
            pl.BlockSpec((h1, d2), lambda f, j: (0, 0)),
            pl.BlockSpec((1, d2), lambda f, j: (0, 0)),
        ],
        out_specs=pl.BlockSpec((1, d2, block_rows), lambda f, j: (f, 0, j)),
        out_shape=jax.ShapeDtypeStruct((fields, d2, batch), jnp.float32),
    )(g, w1, b1.reshape(1, h1), w2, b2.reshape(1, d2))
    return out


def kernel(x, table, W1, b1, W2, b2):
    batch, fields = x.shape
    idx_flat = x.T.reshape(-1).astype(jnp.int32)
    gathered = _sc_gather(table, idx_flat)
    out_t = _tc_mlp_t(gathered, W1, b1, W2, b2, fields, batch)
    return out_t.transpose(2, 0, 1)


# speedup vs baseline: 1.8291x; 1.0027x over previous
import functools

import jax
import jax.numpy as jnp
from jax import lax
from jax.experimental import pallas as pl
from jax.experimental.pallas import tpu as pltpu
from jax.experimental.pallas import tpu_sc as plsc

_NUM_WORKERS = 32
_CHUNK = 128
_NBUF = 8


def _sc_transpose(table_t, nsem=8):
    d, vocab = table_t.shape
    slab = vocab // _NUM_WORKERS
    assert slab * _NUM_WORKERS == vocab

    mesh = plsc.VectorSubcoreMesh(core_axis_name="c", subcore_axis_name="s")

    @functools.partial(
        pl.kernel,
        out_type=jax.ShapeDtypeStruct((vocab, d), jnp.float32),
        mesh=mesh,
        scratch_types=[pltpu.SemaphoreType.DMA] * nsem,
        compiler_params=pltpu.CompilerParams(use_tc_tiling_on_sc=False),
    )
    def transpose_kernel(tt_hbm, out_hbm, *sems):
        wid = lax.axis_index("c") * 16 + lax.axis_index("s")
        v0 = wid * slab

        def dma(e):
            return pltpu.make_async_copy(
                tt_hbm.at[e, pl.ds(v0, slab)],
                out_hbm.at[pl.ds(v0, slab), e],
                sems[e % nsem])

        for e in range(d):
            dma(e).start()
        for e in range(d):
            dma(e).wait()

    return transpose_kernel(table_t)


def _sc_gather(table, idx_flat):
    n = idx_flat.shape[0]
    d = table.shape[1]
    per_worker = _CHUNK * _NBUF
    n_pad = ((n + _NUM_WORKERS * per_worker - 1)
             // (_NUM_WORKERS * per_worker)) * (_NUM_WORKERS * per_worker)
    idx_pad = jnp.zeros((n_pad,), jnp.int32).at[:n].set(idx_flat)
    idx2d = idx_pad.reshape(n_pad // _CHUNK, _CHUNK)
    nch = (n_pad // _CHUNK) // _NUM_WORKERS

    mesh = plsc.VectorSubcoreMesh(core_axis_name="c", subcore_axis_name="s")

    @functools.partial(
        pl.kernel,
        out_type=jax.ShapeDtypeStruct((n_pad, d), jnp.float32),
        mesh=mesh,
        scratch_types=(
            [pltpu.VMEM((nch, _CHUNK), jnp.int32),
             pltpu.VMEM((_NBUF, _CHUNK, d), jnp.float32)]
            + [pltpu.SemaphoreType.DMA] * _NBUF
        ),
        compiler_params=pltpu.CompilerParams(use_tc_tiling_on_sc=False),
    )
    def gather_kernel(table_hbm, idx_hbm, out_hbm, idx_v, rows_v, *gsems):
        wid = lax.axis_index("c") * 16 + lax.axis_index("s")
        chunk0 = wid * nch
        row0 = chunk0 * _CHUNK
        pltpu.sync_copy(idx_hbm.at[pl.ds(chunk0, nch)], idx_v)

        def fire(j, b):
            pltpu.make_async_copy(
                table_hbm.at[idx_v.at[j]], rows_v.at[b], gsems[b]).start()

        for b in range(_NBUF):
            fire(b, b)

        def group(g, _):
            for b in range(_NBUF):
                j = g * _NBUF + b
                pltpu.make_async_copy(
                    table_hbm.at[idx_v.at[j]], rows_v.at[b], gsems[b]).wait()
                pltpu.sync_copy(rows_v.at[b],
                                out_hbm.at[pl.ds(row0 + j * _CHUNK, _CHUNK)])

                @pl.when(j + _NBUF < nch)
                def _():
                    fire(j + _NBUF, b)
            return 0

        lax.fori_loop(0, nch // _NBUF, group, 0)

    return gather_kernel(table, idx2d)[:n]


def _mlp_block_t(h_ref, w1_ref, b1_ref, w2_ref, b2_ref, o_ref):
    h = h_ref[...]
    z = jnp.dot(h, w1_ref[...],
                preferred_element_type=jnp.float32) + b1_ref[...]
    z = jnp.maximum(z, 0.0)
    o = jnp.dot(z, w2_ref[...],
                preferred_element_type=jnp.float32) + b2_ref[...]
    o_ref[0] = o.T


def _tc_mlp_t(g, w1, b1, w2, b2, fields, batch, block_rows=4096):
    d = g.shape[1]
    h1 = w1.shape[1]
    d2 = w2.shape[1]
    nb = batch // block_rows
    out = pl.pallas_call(
        _mlp_block_t,
        grid=(fields, nb),
        in_specs=[
            pl.BlockSpec((block_rows, d), lambda f, j: (f * nb + j, 0)),
            pl.BlockSpec((d, h1), lambda f, j: (0, 0)),
            pl.BlockSpec((1, h1), lambda f, j: (0, 0)),
            pl.BlockSpec((h1, d2), lambda f, j: (0, 0)),
            pl.BlockSpec((1, d2), lambda f, j: (0, 0)),
        ],
        out_specs=pl.BlockSpec((1, d2, block_rows), lambda f, j: (f, 0, j)),
        out_shape=jax.ShapeDtypeStruct((fields, d2, batch), jnp.float32),
    )(g, w1, b1.reshape(1, h1), w2, b2.reshape(1, d2))
    return out


def kernel(x, table, W1, b1, W2, b2):
    batch, fields = x.shape
    idx_flat = x.T.reshape(-1).astype(jnp.int32)
    gathered = _sc_gather(table, idx_flat)
    out_t = _tc_mlp_t(gathered, W1, b1, W2, b2, fields, batch)
    return out_t.transpose(2, 0, 1)


# speedup vs baseline: 2.1015x; 1.1490x over previous
import functools

import jax
import jax.numpy as jnp
from jax import lax
from jax.experimental import pallas as pl
from jax.experimental.pallas import tpu as pltpu
from jax.experimental.pallas import tpu_sc as plsc

_NUM_WORKERS = 32
_CHUNK = 128
_NBUF = 8


def _sc_transpose(table_t, nsem=8):
    d, vocab = table_t.shape
    slab = vocab // _NUM_WORKERS
    assert slab * _NUM_WORKERS == vocab

    mesh = plsc.VectorSubcoreMesh(core_axis_name="c", subcore_axis_name="s")

    @functools.partial(
        pl.kernel,
        out_type=jax.ShapeDtypeStruct((vocab, d), jnp.float32),
        mesh=mesh,
        scratch_types=[pltpu.SemaphoreType.DMA] * nsem,
        compiler_params=pltpu.CompilerParams(use_tc_tiling_on_sc=False),
    )
    def transpose_kernel(tt_hbm, out_hbm, *sems):
        wid = lax.axis_index("c") * 16 + lax.axis_index("s")
        v0 = wid * slab

        def dma(e):
            return pltpu.make_async_copy(
                tt_hbm.at[e, pl.ds(v0, slab)],
                out_hbm.at[pl.ds(v0, slab), e],
                sems[e % nsem])

        for e in range(d):
            dma(e).start()
        for e in range(d):
            dma(e).wait()

    return transpose_kernel(table_t)


def _sc_gather(table, idx2d):
    n_pad = idx2d.shape[0] * _CHUNK
    d = table.shape[1]
    nch = idx2d.shape[0] // _NUM_WORKERS

    mesh = plsc.VectorSubcoreMesh(core_axis_name="c", subcore_axis_name="s")

    @functools.partial(
        pl.kernel,
        out_type=jax.ShapeDtypeStruct((n_pad, d), jnp.float32),
        mesh=mesh,
        scratch_types=(
            [pltpu.VMEM((nch, _CHUNK), jnp.int32),
             pltpu.VMEM((_NBUF, _CHUNK, d), jnp.float32)]
            + [pltpu.SemaphoreType.DMA] * _NBUF
        ),
        compiler_params=pltpu.CompilerParams(use_tc_tiling_on_sc=False),
    )
    def gather_kernel(table_hbm, idx_hbm, out_hbm, idx_v, rows_v, *gsems):
        wid = lax.axis_index("c") * 16 + lax.axis_index("s")
        chunk0 = wid * nch
        row0 = chunk0 * _CHUNK
        pltpu.sync_copy(idx_hbm.at[pl.ds(chunk0, nch)], idx_v)

        def fire(j, b):
            pltpu.make_async_copy(
                table_hbm.at[idx_v.at[j]], rows_v.at[b], gsems[b]).start()

        for b in range(_NBUF):
            fire(b, b)

        def group(g, _):
            for b in range(_NBUF):
                j = g * _NBUF + b
                pltpu.make_async_copy(
                    table_hbm.at[idx_v.at[j]], rows_v.at[b], gsems[b]).wait()
                pltpu.sync_copy(rows_v.at[b],
                                out_hbm.at[pl.ds(row0 + j * _CHUNK, _CHUNK)])

                @pl.when(j + _NBUF < nch)
                def _():
                    fire(j + _NBUF, b)
            return 0

        lax.fori_loop(0, nch // _NBUF, group, 0)

    return gather_kernel(table, idx2d)


def _mlp_block_t(h_ref, w1_ref, b1_ref, w2_ref, b2_ref, o_ref):
    h2 = h_ref[...]
    bm = h2.shape[0]
    d = h2.shape[1] // 2
    for half in range(2):
        h = h2[:, half * d:(half + 1) * d]
        z = jnp.dot(h, w1_ref[...],
                    preferred_element_type=jnp.float32) + b1_ref[...]
        z = jnp.maximum(z, 0.0)
        o = jnp.dot(z, w2_ref[...],
                    preferred_element_type=jnp.float32) + b2_ref[...]
        o_ref[0, :, half * bm:(half + 1) * bm] = o.T


def _tc_mlp_t(g2, w1, b1, w2, b2, fields, batch, block_rows=4096):
    d = g2.shape[1] // 2
    h1 = w1.shape[1]
    d2 = w2.shape[1]
    nb = batch // block_rows
    bm = block_rows // 2
    out = pl.pallas_call(
        _mlp_block_t,
        grid=(fields, nb),
        in_specs=[
            pl.BlockSpec((bm, 2 * d), lambda f, j: (f * nb + j, 0)),
            pl.BlockSpec((d, h1), lambda f, j: (0, 0)),
            pl.BlockSpec((1, h1), lambda f, j: (0, 0)),
            pl.BlockSpec((h1, d2), lambda f, j: (0, 0)),
            pl.BlockSpec((1, d2), lambda f, j: (0, 0)),
        ],
        out_specs=pl.BlockSpec((1, d2, block_rows), lambda f, j: (f, 0, j)),
        out_shape=jax.ShapeDtypeStruct((fields, d2, batch), jnp.float32),
    )(g2, w1, b1.reshape(1, h1), w2, b2.reshape(1, d2))
    return out


def kernel(x, table, W1, b1, W2, b2):
    batch, fields = x.shape
    n = batch * fields
    idx2d = (x.T.reshape(n // 4096, 2, 2048)
             .transpose(0, 2, 1)
             .reshape(n // _CHUNK, _CHUNK).astype(jnp.int32))
    gathered = _sc_gather(table, idx2d)
    g2 = gathered.reshape(n // 2, 128)
    out_t = _tc_mlp_t(g2, W1, b1, W2, b2, fields, batch)
    return out_t.transpose(2, 0, 1)


# speedup vs baseline: 2.2251x; 1.0588x over previous
import functools

import jax
import jax.numpy as jnp
from jax import lax
from jax.experimental import pallas as pl
from jax.experimental.pallas import tpu as pltpu
from jax.experimental.pallas import tpu_sc as plsc

_NUM_WORKERS = 32
_CHUNK = 128
_NBUF = 8


def _sc_transpose(table_t, nsem=8):
    d, vocab = table_t.shape
    slab = vocab // _NUM_WORKERS
    assert slab * _NUM_WORKERS == vocab

    mesh = plsc.VectorSubcoreMesh(core_axis_name="c", subcore_axis_name="s")

    @functools.partial(
        pl.kernel,
        out_type=jax.ShapeDtypeStruct((vocab, d), jnp.float32),
        mesh=mesh,
        scratch_types=[pltpu.SemaphoreType.DMA] * nsem,
        compiler_params=pltpu.CompilerParams(use_tc_tiling_on_sc=False),
    )
    def transpose_kernel(tt_hbm, out_hbm, *sems):
        wid = lax.axis_index("c") * 16 + lax.axis_index("s")
        v0 = wid * slab

        def dma(e):
            return pltpu.make_async_copy(
                tt_hbm.at[e, pl.ds(v0, slab)],
                out_hbm.at[pl.ds(v0, slab), e],
                sems[e % nsem])

        for e in range(d):
            dma(e).start()
        for e in range(d):
            dma(e).wait()

    return transpose_kernel(table_t)


def _sc_gather(table, xt):
    fields, batch = xt.shape
    n = fields * batch
    d = table.shape[1]
    nchunks = n // _CHUNK
    nch = nchunks // _NUM_WORKERS
    bpf = batch // 4096

    mesh = plsc.VectorSubcoreMesh(core_axis_name="c", subcore_axis_name="s")

    @functools.partial(
        pl.kernel,
        out_type=jax.ShapeDtypeStruct((n, d), jnp.float32),
        mesh=mesh,
        scratch_types=(
            [pltpu.VMEM((nch, _CHUNK), jnp.int32),
             pltpu.VMEM((_NBUF, _CHUNK), jnp.int32),
             pltpu.VMEM((_NBUF, _CHUNK, d), jnp.float32)]
            + [pltpu.SemaphoreType.DMA] * (3 * _NBUF)
        ),
        compiler_params=pltpu.CompilerParams(use_tc_tiling_on_sc=False),
    )
    def gather_kernel(table_hbm, xt_hbm, out_hbm, idx_v, pos_v, rows_v, *sems):
        gsems = sems[:_NBUF]
        wsems = sems[_NBUF:2 * _NBUF]
        isems = sems[2 * _NBUF:]
        wid = lax.axis_index("c") * 16 + lax.axis_index("s")
        chunk0 = wid * nch

        def idx_copies(j, si):
            c = chunk0 + j
            block = c // 32
            cc = c - block * 32
            f = block // bpf
            col0 = (block - f * bpf) * 4096 + cc * 64
            row = idx_v.at[j]
            return (
                pltpu.make_async_copy(
                    xt_hbm.at[f, pl.ds(col0, 64)],
                    row.at[pl.ds(0, 64)], isems[si]),
                pltpu.make_async_copy(
                    xt_hbm.at[f, pl.ds(col0 + 2048, 64)],
                    row.at[pl.ds(64, 64)], isems[si]),
            )

        def fire_idx(j, si):
            for cp in idx_copies(j, si):
                cp.start()

        def wait_idx(j, si):
            for cp in idx_copies(j, si):
                cp.wait()

        def fire_gather(j, b):
            pltpu.make_async_copy(
                table_hbm.at[idx_v.at[j]], rows_v.at[b], gsems[b]).start()

        lanes = lax.broadcasted_iota(jnp.int32, (16,), 0)

        for k in range(2 * _NBUF):
            fire_idx(k, k % _NBUF)
        for b in range(_NBUF):
            wait_idx(b, b)
            fire_gather(b, b)

        def group(g, _):
            for b in range(_NBUF):
                j = g * _NBUF + b
                pltpu.make_async_copy(
                    table_hbm.at[idx_v.at[j]], rows_v.at[b], gsems[b]).wait()
                row0 = (chunk0 + j) * _CHUNK
                pos_row = pos_v.at[b]
                for k in range(8):
                    base = row0 + 32 * (k % 4) + (k // 4)
                    pos_row[pl.ds(16 * k, 16)] = base + 2 * lanes
                pltpu.make_async_copy(
                    rows_v.at[b], out_hbm.at[pos_v.at[b]], wsems[b]).start()

                @pl.when(j + _NBUF < nch)
                def _():
                    pltpu.make_async_copy(
                        rows_v.at[b], out_hbm.at[pos_v.at[b]],
                        wsems[b]).wait()
                    wait_idx(j + _NBUF, b)
                    fire_gather(j + _NBUF, b)

                @pl.when(j + 2 * _NBUF < nch)
                def _():
                    fire_idx(j + 2 * _NBUF, b)

                @pl.when(j + _NBUF >= nch)
                def _():
                    pltpu.make_async_copy(
                        rows_v.at[b], out_hbm.at[pos_v.at[b]],
                        wsems[b]).wait()
            return 0

        lax.fori_loop(0, nch // _NBUF, group, 0)

    return gather_kernel(table, xt)


def _mlp_block_t(h_ref, w1_ref, b1_ref, w2_ref, b2_ref, o_ref):
    h2 = h_ref[...]
    bm = h2.shape[0]
    d = h2.shape[1] // 2
    for half in range(2):
        h = h2[:, half * d:(half + 1) * d]
        z = jnp.dot(h, w1_ref[...],
                    preferred_element_type=jnp.float32) + b1_ref[...]
        z = jnp.maximum(z, 0.0)
        o = jnp.dot(z, w2_ref[...],
                    preferred_element_type=jnp.float32) + b2_ref[...]
        o_ref[0, :, half * bm:(half + 1) * bm] = o.T


def _tc_mlp_t(g2, w1, b1, w2, b2, fields, batch, block_rows=4096):
    d = g2.shape[1] // 2
    h1 = w1.shape[1]
    d2 = w2.shape[1]
    nb = batch // block_rows
    bm = block_rows // 2
    out = pl.pallas_call(
        _mlp_block_t,
        grid=(fields, nb),
        in_specs=[
            pl.BlockSpec((bm, 2 * d), lambda f, j: (f * nb + j, 0)),
            pl.BlockSpec((d, h1), lambda f, j: (0, 0)),
            pl.BlockSpec((1, h1), lambda f, j: (0, 0)),
            pl.BlockSpec((h1, d2), lambda f, j: (0, 0)),
            pl.BlockSpec((1, d2), lambda f, j: (0, 0)),
        ],
        out_specs=pl.BlockSpec((1, d2, block_rows), lambda f, j: (f, 0, j)),
        out_shape=jax.ShapeDtypeStruct((fields, d2, batch), jnp.float32),
    )(g2, w1, b1.reshape(1, h1), w2, b2.reshape(1, d2))
    return out


def kernel(x, table, W1, b1, W2, b2):
    batch, fields = x.shape
    n = batch * fields
    gathered = _sc_gather(table, x.T.astype(jnp.int32))
    g2 = gathered.reshape(n // 2, 128)
    out_t = _tc_mlp_t(g2, W1, b1, W2, b2, fields, batch)
    return out_t.transpose(2, 0, 1)
